# baseline (speedup 1.0000x reference)
import functools

import jax
import jax.numpy as jnp
from jax import lax
from jax.experimental import pallas as pl
from jax.experimental.pallas import tpu as pltpu
from jax.experimental.pallas import tpu_sc as plsc

N = 4096
E = 65536
K = 4096
NT = 16
EPT = E // NT
CAP = 512
LOCCAP = 2048
NSLOT = 4
ROWS = 2
SLAB = N // 32
LOG2_N = 12
VBIT = 1 << 24
CMASK = 0x00FFFFFF
ABSM = 0x7FFFFFFF
THETA_BITS = 1057791828
SH_FRAG1 = 4 * NT * 256
SH_FRAG2 = SH_FRAG1 + NT * CAP
SH_CNT = SH_FRAG2 + NT * CAP
SH_TOTAL = SH_CNT + NT * 16


def _iota16():
    return lax.iota(jnp.int32, 16)


def _popcount(mask):
    return jnp.max(jnp.cumsum(mask.astype(jnp.int32)))


def _compact_dest(off, mask, dump_base):
    inc = jnp.cumsum(mask.astype(jnp.int32))
    dest = jnp.where(mask, off + inc - 1, dump_base + _iota16())
    return dest, off + jnp.max(inc)


def _fused_body(keys_hbm, rc_hbm, adj_hbm, out_hbm,
                keys_v, rc_v, hist_v, hmerge_v, mrg_v,
                l1_v, l2_v, e1_v, e2_v, loc_v, buf_v, hist_sh,
                sin0, sin1, sin2, sin3, sout0, sout1, sout2, sout3):
    core = lax.axis_index("c")
    tile = lax.axis_index("s")
    w = core * 16 + tile
    lo = w * SLAB

    nch = SLAB // ROWS
    sins = (sin0, sin1, sin2, sin3)
    souts = (sout0, sout1, sout2, sout3)

    def _start_in(ch, slot):
        for s in range(NSLOT):
            @pl.when(slot == s)
            def _():
                pltpu.async_copy(
                    adj_hbm.at[pl.ds(lo + ch * ROWS, ROWS), :],
                    buf_v.at[pl.ds(s * ROWS, ROWS), :], sins[s])

    def _wait_in(slot):
        for s in range(NSLOT):
            @pl.when(slot == s)
            def _():
                pltpu.make_async_copy(
                    adj_hbm.at[pl.ds(0, ROWS), :],
                    buf_v.at[pl.ds(s * ROWS, ROWS), :], sins[s]).wait()

    def _start_out(ch, slot):
        for s in range(NSLOT):
            @pl.when(slot == s)
            def _():
                pltpu.async_copy(
                    buf_v.at[pl.ds(s * ROWS, ROWS), :],
                    out_hbm.at[pl.ds(lo + ch * ROWS, ROWS), :], souts[s])

    def _wait_out(slot):
        for s in range(NSLOT):
            @pl.when(slot == s)
            def _():
                pltpu.make_async_copy(
                    buf_v.at[pl.ds(s * ROWS, ROWS), :],
                    out_hbm.at[pl.ds(0, ROWS), :], souts[s]).wait()

    for p in range(NSLOT - 1):
        _start_in(p, jnp.int32(p))

    base = tile * EPT
    pltpu.sync_copy(keys_hbm.at[pl.ds(base, EPT)], keys_v)
    pltpu.sync_copy(rc_hbm.at[pl.ds(base, EPT)], rc_v)

    def _round(rnd, carry):
        t_prefix, k_rem = carry
        shift = 24 - 8 * rnd
        def _z(i, _):
            hist_v[pl.ds(i * 16, 16)] = jnp.zeros((16,), jnp.int32)
            return 0
        lax.fori_loop(0, 16, _z, 0)

        ones = jnp.ones((16,), jnp.int32)
        def _h(i, _):
            key = keys_v[pl.ds(i * 16, 16)] & ABSM
            act = jnp.where(
                rnd == 0,
                jnp.ones((16,), jnp.bool_),
                (key >> (shift + 8)) == (t_prefix >> (shift + 8)))
            b = (key >> shift) & jnp.int32(0xFF)
            b = jnp.where(act, b, 256 + _iota16())
            plsc.addupdate_scatter(hist_v, [b], ones)
            return 0
        lax.fori_loop(0, EPT // 16, _h, 0)

        pltpu.sync_copy(hist_v.at[pl.ds(0, 256)],
                        hist_sh.at[pl.ds(rnd * (NT * 256) + tile * 256,
                                         256)])
        plsc.subcore_barrier()
        pltpu.sync_copy(hist_sh.at[pl.ds(rnd * (NT * 256), NT * 256)],
                        hmerge_v)
        def _m(l, _):
            def _mt(t, acc):
                return acc + hmerge_v[pl.ds(t * 256 + l * 16, 16)]
            mrg_v[pl.ds(l * 16, 16)] = lax.fori_loop(
                0, NT, _mt, jnp.zeros((16,), jnp.int32))
            return 0
        lax.fori_loop(0, 16, _m, 0)

        def _scan(jj, sc):
            k_r, above, found, bstar = sc
            j = 15 - jj
            v = mrg_v[pl.ds(j * 16, 16)]
            sfx = lax.rev(jnp.cumsum(lax.rev(v, (0,))), (0,))
            incl = above + sfx
            tot = jnp.max(sfx)
            hit = jnp.logical_and(jnp.logical_not(found),
                                  above + tot >= k_r)
            msk = incl >= k_r
            cnt = _popcount(msk)
            lane = cnt - 1
            strictly_above = jnp.max(
                jnp.where(_iota16() == lane, incl - v, 0))
            b_hit = j * 16 + lane
            k_r2 = jnp.where(hit, k_r - strictly_above, k_r)
            bstar2 = jnp.where(hit, b_hit, bstar)
            return (k_r2, above + tot, jnp.logical_or(found, hit), bstar2)
        k_rem2, _, _, bstar = lax.fori_loop(
            0, 16, _scan,
            (k_rem, jnp.int32(0), jnp.bool_(False), jnp.int32(0)))
        return (t_prefix | (bstar << shift), k_rem2)

    t_key, m_eq = lax.fori_loop(0, 4, _round, (jnp.int32(0), jnp.int32(K)))

    b4 = t_key & jnp.int32(0xFF)
    eq_counts = plsc.load_gather(hmerge_v, [_iota16() * 256 + b4])
    prefix_before = jnp.sum(jnp.where(_iota16() < tile, eq_counts, 0))

    def _sent(i, _):
        l1_v[pl.ds(i * 16, 16)] = jnp.full((16,), -1, jnp.int32)
        l2_v[pl.ds(i * 16, 16)] = jnp.full((16,), -1, jnp.int32)
        return 0
    lax.fori_loop(0, CAP // 16, _sent, 0)

    def _emit(i, carry):
        off, eqseen = carry
        kf = keys_v[pl.ds(i * 16, 16)]
        key = kf & ABSM
        neg = kf < 0
        gt = key > t_key
        eq = key == t_key
        eqc = jnp.cumsum(eq.astype(jnp.int32))
        rank = prefix_before + eqseen + eqc - 1
        keep = jnp.logical_or(gt, jnp.logical_and(eq, rank < m_eq))
        big = key > jnp.int32(THETA_BITS)
        sig = jnp.logical_and(keep, big)
        sigp = jnp.logical_and(sig, jnp.logical_not(neg))
        vbit = sigp.astype(jnp.int32) << 24
        rc = rc_v[pl.ds(i * 16, 16)]
        e1 = rc | vbit
        e2 = (((rc & jnp.int32(N - 1)) << LOG2_N) | (rc >> LOG2_N)) | vbit
        offc = jnp.minimum(off, CAP - 16)
        dest, off2 = _compact_dest(offc, sig, CAP + 16)
        plsc.store_scatter(l1_v, [dest], e1)
        plsc.store_scatter(l2_v, [dest], e2)
        return (jnp.minimum(off2, jnp.int32(CAP)), eqseen + jnp.max(eqc))
    lax.fori_loop(0, EPT // 16, _emit, (jnp.int32(0), jnp.int32(0)))

    pltpu.sync_copy(l1_v.at[pl.ds(0, CAP)],
                    hist_sh.at[pl.ds(SH_FRAG1 + tile * CAP, CAP)])
    pltpu.sync_copy(l2_v.at[pl.ds(0, CAP)],
                    hist_sh.at[pl.ds(SH_FRAG2 + tile * CAP, CAP)])
    plsc.subcore_barrier()
    pltpu.sync_copy(hist_sh.at[pl.ds(SH_FRAG1, NT * CAP)], e1_v)
    pltpu.sync_copy(hist_sh.at[pl.ds(SH_FRAG2, NT * CAP)], e2_v)

    def _filter(eref, off0):
        def _j(j, off2):
            e = eref[pl.ds(j * 16, 16)]
            row = (e & CMASK) >> LOG2_N
            ins = jnp.logical_and(
                jnp.logical_and(row >= lo, row < lo + SLAB), e != -1)
            le = ((e & CMASK) - lo * N) | (e & VBIT)
            dest, off3 = _compact_dest(jnp.minimum(off2, LOCCAP - 16),
                                       ins, LOCCAP + 16)
            plsc.store_scatter(loc_v, [dest], le)
            return jnp.minimum(off3, jnp.int32(LOCCAP))
        return lax.fori_loop(0, NT * CAP // 16, _j, off0)

    off = _filter(e1_v, jnp.int32(0))
    off = _filter(e2_v, off)
    loc_v[pl.ds(off, 16)] = jnp.full((16,), -1, jnp.int32)
    nloc = (off + 15) // 16

    def _chunk(ch, _):
        slot = lax.rem(ch, NSLOT)
        _wait_in(slot)
        lbase = ch * ROWS * N
        def _ap(v, _2):
            le = loc_v[pl.ds(v * 16, 16)]
            rel = (le & CMASK) - lbase
            inch = jnp.logical_and(
                jnp.logical_and(rel >= 0, rel < ROWS * N), le != -1)
            vv = jnp.where((le & VBIT) != 0, jnp.float32(1.0),
                           jnp.float32(0.0))
            rr = jnp.where(inch, slot * ROWS + (rel >> LOG2_N), NSLOT * ROWS)
            cc = jnp.where(inch, rel & jnp.int32(N - 1), _iota16())
            plsc.store_scatter(buf_v, [rr, cc], vv)
            return 0
        lax.fori_loop(0, nloc, _ap, 0)
        _start_out(ch, slot)
        nslot = lax.rem(ch + NSLOT - 1, NSLOT)

        @pl.when(ch + NSLOT - 1 < nch)
        def _():
            @pl.when(ch >= 1)
            def _():
                _wait_out(nslot)
            _start_in(ch + NSLOT - 1, nslot)
        return 0
    lax.fori_loop(0, nch, _chunk, 0)
    for d in range(NSLOT):
        _wait_out(jnp.int32((nch - NSLOT + d) % NSLOT))


def kernel(adj, M, edge_pairs, top_k):
    del top_k
    mesh = plsc.VectorSubcoreMesh(core_axis_name="c", subcore_axis_name="s")

    fused = functools.partial(
        pl.kernel,
        out_type=jax.ShapeDtypeStruct((N, N), jnp.float32),
        mesh=mesh,
        compiler_params=pltpu.CompilerParams(needs_layout_passes=False),
        scratch_types=[
            pltpu.VMEM((EPT,), jnp.int32),
            pltpu.VMEM((EPT,), jnp.int32),
            pltpu.VMEM((272,), jnp.int32),
            pltpu.VMEM((NT * 256,), jnp.int32),
            pltpu.VMEM((256,), jnp.int32),
            pltpu.VMEM((CAP + 32,), jnp.int32),
            pltpu.VMEM((CAP + 32,), jnp.int32),
            pltpu.VMEM((NT * CAP,), jnp.int32),
            pltpu.VMEM((NT * CAP,), jnp.int32),
            pltpu.VMEM((LOCCAP + 32,), jnp.int32),
            pltpu.VMEM((NSLOT * ROWS + 4, N), jnp.float32),
            pltpu.VMEM_SHARED((SH_TOTAL,), jnp.int32),
            pltpu.SemaphoreType.DMA,
            pltpu.SemaphoreType.DMA,
            pltpu.SemaphoreType.DMA,
            pltpu.SemaphoreType.DMA,
            pltpu.SemaphoreType.DMA,
            pltpu.SemaphoreType.DMA,
            pltpu.SemaphoreType.DMA,
            pltpu.SemaphoreType.DMA,
        ],
    )(_fused_body)

    keys = lax.bitcast_convert_type(M, jnp.int32)
    rc = edge_pairs[:, 0] * N + edge_pairs[:, 1]
    return fused(keys, rc, adj)


# speedup vs baseline: 1.0190x; 1.0190x over previous
import functools

import jax
import jax.numpy as jnp
from jax import lax
from jax.experimental import pallas as pl
from jax.experimental.pallas import tpu as pltpu
from jax.experimental.pallas import tpu_sc as plsc

N = 4096
E = 65536
K = 4096
NT = 16
EPT = E // NT
CAP = 512
LOCCAP = 2048
NSLOT = 3
ROWS = 4
SLAB = N // 32
LOG2_N = 12
VBIT = 1 << 24
CMASK = 0x00FFFFFF
ABSM = 0x7FFFFFFF
THETA_BITS = 1057791828
SH_FRAG1 = 4 * NT * 256
SH_FRAG2 = SH_FRAG1 + NT * CAP
SH_CNT = SH_FRAG2 + NT * CAP
SH_TOTAL = SH_CNT + NT * 16


def _iota16():
    return lax.iota(jnp.int32, 16)


def _popcount(mask):
    return jnp.max(jnp.cumsum(mask.astype(jnp.int32)))


def _compact_dest(off, mask, dump_base):
    inc = jnp.cumsum(mask.astype(jnp.int32))
    dest = jnp.where(mask, off + inc - 1, dump_base + _iota16())
    return dest, off + jnp.max(inc)


def _fused_body(keys_hbm, rc_hbm, adj_hbm, out_hbm,
                keys_v, rc_v, hist_v, hmerge_v, mrg_v,
                l1_v, l2_v, e1_v, e2_v, loc_v, buf_v, hist_sh,
                sin0, sin1, sin2, sin3, sout0, sout1, sout2, sout3):
    core = lax.axis_index("c")
    tile = lax.axis_index("s")
    w = core * 16 + tile
    lo = w * SLAB

    nch = SLAB // ROWS
    sins = (sin0, sin1, sin2, sin3)
    souts = (sout0, sout1, sout2, sout3)

    def _start_in(ch, slot):
        for s in range(NSLOT):
            @pl.when(slot == s)
            def _():
                pltpu.async_copy(
                    adj_hbm.at[pl.ds(lo + ch * ROWS, ROWS), :],
                    buf_v.at[pl.ds(s * ROWS, ROWS), :], sins[s])

    def _wait_in(slot):
        for s in range(NSLOT):
            @pl.when(slot == s)
            def _():
                pltpu.make_async_copy(
                    adj_hbm.at[pl.ds(0, ROWS), :],
                    buf_v.at[pl.ds(s * ROWS, ROWS), :], sins[s]).wait()

    def _start_out(ch, slot):
        for s in range(NSLOT):
            @pl.when(slot == s)
            def _():
                pltpu.async_copy(
                    buf_v.at[pl.ds(s * ROWS, ROWS), :],
                    out_hbm.at[pl.ds(lo + ch * ROWS, ROWS), :], souts[s])

    def _wait_out(slot):
        for s in range(NSLOT):
            @pl.when(slot == s)
            def _():
                pltpu.make_async_copy(
                    buf_v.at[pl.ds(s * ROWS, ROWS), :],
                    out_hbm.at[pl.ds(0, ROWS), :], souts[s]).wait()

    for p in range(NSLOT - 1):
        _start_in(p, jnp.int32(p))

    base = tile * EPT
    pltpu.sync_copy(keys_hbm.at[pl.ds(base, EPT)], keys_v)
    pltpu.sync_copy(rc_hbm.at[pl.ds(base, EPT)], rc_v)

    def _round(rnd, carry):
        t_prefix, k_rem = carry
        shift = 24 - 8 * rnd
        def _z(i, _):
            hist_v[pl.ds(i * 16, 16)] = jnp.zeros((16,), jnp.int32)
            return 0
        lax.fori_loop(0, 16, _z, 0)

        ones = jnp.ones((16,), jnp.int32)
        def _h(i, _):
            key = keys_v[pl.ds(i * 16, 16)] & ABSM
            act = jnp.where(
                rnd == 0,
                jnp.ones((16,), jnp.bool_),
                (key >> (shift + 8)) == (t_prefix >> (shift + 8)))
            b = (key >> shift) & jnp.int32(0xFF)
            b = jnp.where(act, b, 256 + _iota16())
            plsc.addupdate_scatter(hist_v, [b], ones)
            return 0
        lax.fori_loop(0, EPT // 16, _h, 0)

        pltpu.sync_copy(hist_v.at[pl.ds(0, 256)],
                        hist_sh.at[pl.ds(rnd * (NT * 256) + tile * 256,
                                         256)])
        plsc.subcore_barrier()
        pltpu.sync_copy(hist_sh.at[pl.ds(rnd * (NT * 256), NT * 256)],
                        hmerge_v)
        def _m(l, _):
            def _mt(t, acc):
                return acc + hmerge_v[pl.ds(t * 256 + l * 16, 16)]
            mrg_v[pl.ds(l * 16, 16)] = lax.fori_loop(
                0, NT, _mt, jnp.zeros((16,), jnp.int32))
            return 0
        lax.fori_loop(0, 16, _m, 0)

        def _scan(jj, sc):
            k_r, above, found, bstar = sc
            j = 15 - jj
            v = mrg_v[pl.ds(j * 16, 16)]
            sfx = lax.rev(jnp.cumsum(lax.rev(v, (0,))), (0,))
            incl = above + sfx
            tot = jnp.max(sfx)
            hit = jnp.logical_and(jnp.logical_not(found),
                                  above + tot >= k_r)
            msk = incl >= k_r
            cnt = _popcount(msk)
            lane = cnt - 1
            strictly_above = jnp.max(
                jnp.where(_iota16() == lane, incl - v, 0))
            b_hit = j * 16 + lane
            k_r2 = jnp.where(hit, k_r - strictly_above, k_r)
            bstar2 = jnp.where(hit, b_hit, bstar)
            return (k_r2, above + tot, jnp.logical_or(found, hit), bstar2)
        k_rem2, _, _, bstar = lax.fori_loop(
            0, 16, _scan,
            (k_rem, jnp.int32(0), jnp.bool_(False), jnp.int32(0)))
        return (t_prefix | (bstar << shift), k_rem2)

    t_key, m_eq = lax.fori_loop(0, 4, _round, (jnp.int32(0), jnp.int32(K)))

    b4 = t_key & jnp.int32(0xFF)
    eq_counts = plsc.load_gather(hmerge_v, [_iota16() * 256 + b4])
    prefix_before = jnp.sum(jnp.where(_iota16() < tile, eq_counts, 0))

    def _sent(i, _):
        l1_v[pl.ds(i * 16, 16)] = jnp.full((16,), -1, jnp.int32)
        l2_v[pl.ds(i * 16, 16)] = jnp.full((16,), -1, jnp.int32)
        return 0
    lax.fori_loop(0, CAP // 16, _sent, 0)

    def _emit(i, carry):
        off, eqseen = carry
        kf = keys_v[pl.ds(i * 16, 16)]
        key = kf & ABSM
        neg = kf < 0
        gt = key > t_key
        eq = key == t_key
        eqc = jnp.cumsum(eq.astype(jnp.int32))
        rank = prefix_before + eqseen + eqc - 1
        keep = jnp.logical_or(gt, jnp.logical_and(eq, rank < m_eq))
        big = key > jnp.int32(THETA_BITS)
        sig = jnp.logical_and(keep, big)
        sigp = jnp.logical_and(sig, jnp.logical_not(neg))
        vbit = sigp.astype(jnp.int32) << 24
        rc = rc_v[pl.ds(i * 16, 16)]
        e1 = rc | vbit
        e2 = (((rc & jnp.int32(N - 1)) << LOG2_N) | (rc >> LOG2_N)) | vbit
        offc = jnp.minimum(off, CAP - 16)
        dest, off2 = _compact_dest(offc, sig, CAP + 16)
        plsc.store_scatter(l1_v, [dest], e1)
        plsc.store_scatter(l2_v, [dest], e2)
        return (jnp.minimum(off2, jnp.int32(CAP)), eqseen + jnp.max(eqc))
    lax.fori_loop(0, EPT // 16, _emit, (jnp.int32(0), jnp.int32(0)))

    pltpu.sync_copy(l1_v.at[pl.ds(0, CAP)],
                    hist_sh.at[pl.ds(SH_FRAG1 + tile * CAP, CAP)])
    pltpu.sync_copy(l2_v.at[pl.ds(0, CAP)],
                    hist_sh.at[pl.ds(SH_FRAG2 + tile * CAP, CAP)])
    plsc.subcore_barrier()
    pltpu.sync_copy(hist_sh.at[pl.ds(SH_FRAG1, NT * CAP)], e1_v)
    pltpu.sync_copy(hist_sh.at[pl.ds(SH_FRAG2, NT * CAP)], e2_v)

    def _filter(eref, off0):
        def _j(j, off2):
            e = eref[pl.ds(j * 16, 16)]
            row = (e & CMASK) >> LOG2_N
            ins = jnp.logical_and(
                jnp.logical_and(row >= lo, row < lo + SLAB), e != -1)
            le = ((e & CMASK) - lo * N) | (e & VBIT)
            dest, off3 = _compact_dest(jnp.minimum(off2, LOCCAP - 16),
                                       ins, LOCCAP + 16)
            plsc.store_scatter(loc_v, [dest], le)
            return jnp.minimum(off3, jnp.int32(LOCCAP))
        return lax.fori_loop(0, NT * CAP // 16, _j, off0)

    off = _filter(e1_v, jnp.int32(0))
    off = _filter(e2_v, off)
    loc_v[pl.ds(off, 16)] = jnp.full((16,), -1, jnp.int32)
    nloc = (off + 15) // 16

    def _chunk(ch, _):
        slot = lax.rem(ch, NSLOT)
        _wait_in(slot)
        lbase = ch * ROWS * N
        def _ap(v, _2):
            le = loc_v[pl.ds(v * 16, 16)]
            rel = (le & CMASK) - lbase
            inch = jnp.logical_and(
                jnp.logical_and(rel >= 0, rel < ROWS * N), le != -1)
            vv = jnp.where((le & VBIT) != 0, jnp.float32(1.0),
                           jnp.float32(0.0))
            rr = jnp.where(inch, slot * ROWS + (rel >> LOG2_N), NSLOT * ROWS)
            cc = jnp.where(inch, rel & jnp.int32(N - 1), _iota16())
            plsc.store_scatter(buf_v, [rr, cc], vv)
            return 0
        lax.fori_loop(0, nloc, _ap, 0)
        _start_out(ch, slot)
        nslot = lax.rem(ch + NSLOT - 1, NSLOT)

        @pl.when(ch + NSLOT - 1 < nch)
        def _():
            @pl.when(ch >= 1)
            def _():
                _wait_out(nslot)
            _start_in(ch + NSLOT - 1, nslot)
        return 0
    lax.fori_loop(0, nch, _chunk, 0)
    for d in range(NSLOT):
        _wait_out(jnp.int32((nch - NSLOT + d) % NSLOT))


def kernel(adj, M, edge_pairs, top_k):
    del top_k
    mesh = plsc.VectorSubcoreMesh(core_axis_name="c", subcore_axis_name="s")

    fused = functools.partial(
        pl.kernel,
        out_type=jax.ShapeDtypeStruct((N, N), jnp.float32),
        mesh=mesh,
        compiler_params=pltpu.CompilerParams(needs_layout_passes=False),
        scratch_types=[
            pltpu.VMEM((EPT,), jnp.int32),
            pltpu.VMEM((EPT,), jnp.int32),
            pltpu.VMEM((272,), jnp.int32),
            pltpu.VMEM((NT * 256,), jnp.int32),
            pltpu.VMEM((256,), jnp.int32),
            pltpu.VMEM((CAP + 32,), jnp.int32),
            pltpu.VMEM((CAP + 32,), jnp.int32),
            pltpu.VMEM((NT * CAP,), jnp.int32),
            pltpu.VMEM((NT * CAP,), jnp.int32),
            pltpu.VMEM((LOCCAP + 32,), jnp.int32),
            pltpu.VMEM((NSLOT * ROWS + 4, N), jnp.float32),
            pltpu.VMEM_SHARED((SH_TOTAL,), jnp.int32),
            pltpu.SemaphoreType.DMA,
            pltpu.SemaphoreType.DMA,
            pltpu.SemaphoreType.DMA,
            pltpu.SemaphoreType.DMA,
            pltpu.SemaphoreType.DMA,
            pltpu.SemaphoreType.DMA,
            pltpu.SemaphoreType.DMA,
            pltpu.SemaphoreType.DMA,
        ],
    )(_fused_body)

    keys = lax.bitcast_convert_type(M, jnp.int32)
    rc = edge_pairs[:, 0] * N + edge_pairs[:, 1]
    return fused(keys, rc, adj)
